# Initial kernel scaffold; baseline (speedup 1.0000x reference)
import jax
import jax.numpy as jnp
from jax.experimental import pallas as pl


def kernel(text, table, W1, b1, W2, b2):
    raise NotImplementedError("write your pallas kernel here")


# speedup vs baseline: 16.3744x; 16.3744x over previous
import functools

import jax
import jax.numpy as jnp
from jax import lax
from jax.experimental import pallas as pl
from jax.experimental.pallas import tpu as pltpu
from jax.experimental.pallas import tpu_sc as plsc

_B = 4096
_L = 200
_D = 128
_VPR = _D // 16
_CHUNK = 100
_NBUF = 4


def _pool_kernel_body(text_hbm, table_hbm, pooled_hbm, idx_v, buf_v, out_v,
                      *sems):
    nc = 2
    wid = lax.axis_index("s") * nc + lax.axis_index("c")
    n_samples = _B // 32
    n_chunks = n_samples * (_L // _CHUNK)

    pltpu.sync_copy(text_hbm.at[pl.ds(wid * n_chunks, n_chunks)], idx_v)

    def gather_start(chunk, b):
        pltpu.async_copy(table_hbm.at[idx_v.at[chunk]], buf_v.at[b], sems[b])

    def gather_wait(chunk, b):
        pltpu.make_async_copy(
            table_hbm.at[idx_v.at[chunk]], buf_v.at[b], sems[b]).wait()

    def accum_chunk(bref):
        def body(r, accs):
            return tuple(a + bref[r, pl.ds(16 * v, 16)]
                         for v, a in enumerate(accs))
        init = tuple(jnp.zeros((16,), jnp.float32) for _ in range(_VPR))
        return lax.fori_loop(0, _CHUNK, body, init)

    for b in range(_NBUF):
        gather_start(b, b)

    def outer(g, carry):
        for half in range(2):
            s = 2 * g + half
            accs = None
            for j in range(2):
                b = 2 * half + j
                chunk = _NBUF * g + b
                gather_wait(chunk, b)
                a = accum_chunk(buf_v.at[b])
                accs = a if accs is None else tuple(
                    x + y for x, y in zip(accs, a))
                nxt = chunk + _NBUF

                @pl.when(nxt < n_chunks)
                def _():
                    gather_start(nxt, b)
            for v in range(_VPR):
                out_v[s, pl.ds(16 * v, 16)] = accs[v]
        return carry

    lax.fori_loop(0, n_chunks // _NBUF, outer, 0)
    pltpu.sync_copy(out_v, pooled_hbm.at[pl.ds(wid * n_samples, n_samples)])


def _make_pool_kernel():
    mesh = plsc.VectorSubcoreMesh(core_axis_name="c", subcore_axis_name="s")
    n_samples = _B // 32
    n_chunks = n_samples * (_L // _CHUNK)
    return pl.kernel(
        _pool_kernel_body,
        out_type=jax.ShapeDtypeStruct((_B, _D), jnp.float32),
        mesh=mesh,
        scratch_types=[
            pltpu.VMEM((n_chunks, _CHUNK), jnp.int32),
            pltpu.VMEM((_NBUF, _CHUNK, _D), jnp.float32),
            pltpu.VMEM((n_samples, _D), jnp.float32),
        ] + [pltpu.SemaphoreType.DMA] * _NBUF,
    )


def _mlp_body(p_ref, a_ref, b1_ref, w2_ref, b2_ref, o_ref):
    h = jnp.dot(p_ref[...], a_ref[...], preferred_element_type=jnp.float32)
    h = jnp.maximum(h + b1_ref[...], 0.0)
    o_ref[...] = (jnp.dot(h, w2_ref[...], preferred_element_type=jnp.float32)
                  + b2_ref[...])


def kernel(text, table, W1, b1, W2, b2):
    text_flat = text.reshape(_B * (_L // _CHUNK), _CHUNK)
    pooled = _make_pool_kernel()(text_flat, table)
    a = W1.T * (1.0 / _L)
    out = pl.pallas_call(
        _mlp_body,
        out_shape=jax.ShapeDtypeStruct((_B, W2.shape[0]), jnp.float32),
    )(pooled, a, b1.reshape(1, -1), W2.T, b2.reshape(1, -1))
    return out
